# baseline (speedup 1.0000x reference)
import functools

import jax
import jax.numpy as jnp
from jax.experimental import pallas as pl
from jax.experimental.pallas import tpu as pltpu
from jax.experimental.pallas import tpu_sc as plsc

MASK_ID = 5
K = 3
EPS = 1e-06
NEG_INF = float("-inf")
I32_BIG = jnp.iinfo(jnp.int32).max

CH = 10000
NCH = 10
NVEC = CH // 16
U1 = 25
GV = 25
NG = NVEC // GV
RES_W = 16


def _bmax_splat(v):
    i = jax.lax.broadcasted_iota(jnp.int32, (16,), 0)
    for d in (1, 2, 4, 8):
        v = jnp.maximum(v, v[i ^ d])
    return v


def _bmin_splat_i32(v):
    i = jax.lax.broadcasted_iota(jnp.int32, (16,), 0)
    for d in (1, 2, 4, 8):
        v = jnp.minimum(v, v[i ^ d])
    return v


def _sc_topk_body(x_hbm, topv_hbm, topi_hbm, buf0, buf1, resv, resi,
                  sem0, sem1, *, rows_pw, nw):
    nc = 2
    wid = jax.lax.axis_index("s") * nc + jax.lax.axis_index("c")
    iota = jax.lax.broadcasted_iota(jnp.int32, (16,), 0)
    neg16 = jnp.full((16,), NEG_INF, jnp.float32)
    big16 = jnp.full((16,), I32_BIG, jnp.int32)
    bufs = (buf0, buf1)
    sems = (sem0, sem1)

    def extract_and_merge(buf, c, g, st):
        b0 = c * CH + g * (GV * 16)
        lb = g * (GV * 16)
        rem1 = jnp.int32(-1)
        rem2 = jnp.int32(-1)
        cand = []
        for _ in range(K):
            def maxpass(i, m, rem1=rem1, rem2=rem2):
                v = buf[pl.ds(lb + i * 16, 16)]
                gi = (b0 + i * 16) + iota
                keep = (gi != rem1) & (gi != rem2)
                return jnp.maximum(m, jnp.where(keep, v, NEG_INF))

            mvec = jax.lax.fori_loop(0, GV, maxpass, neg16)
            mv = _bmax_splat(mvec)[0]

            def ipass(i, iv, mv=mv, rem1=rem1, rem2=rem2):
                v = buf[pl.ds(lb + i * 16, 16)]
                gi = (b0 + i * 16) + iota
                ok = (v == mv) & (gi != rem1) & (gi != rem2)
                return jnp.minimum(iv, jnp.where(ok, gi, I32_BIG))

            ivec = jax.lax.fori_loop(0, GV, ipass, big16)
            mi = _bmin_splat_i32(ivec)[0]
            cand.append((mv, mi))
            rem2 = rem1
            rem1 = mi

        vs = [st[0], st[1], st[2], cand[0][0], cand[1][0], cand[2][0]]
        ix = [st[3], st[4], st[5], cand[0][1], cand[1][1], cand[2][1]]
        outv, outi = [], []
        for _ in range(K):
            bv, bi = vs[0], ix[0]
            for t in range(1, 6):
                better = (vs[t] > bv) | ((vs[t] == bv) & (ix[t] < bi))
                bv = jnp.where(better, vs[t], bv)
                bi = jnp.where(better, ix[t], bi)
            outv.append(bv)
            outi.append(bi)
            vs = [jnp.where((vs[t] == bv) & (ix[t] == bi),
                            jnp.float32(NEG_INF), vs[t]) for t in range(6)]
        return (outv[0], outv[1], outv[2], outi[0], outi[1], outi[2])

    def process_chunk(buf, c, st):
        def l1(i, m):
            mm = m
            for u in range(U1):
                mm = jnp.maximum(mm, buf[pl.ds((i * U1 + u) * 16, 16)])
            return mm

        m_c = jax.lax.fori_loop(0, NVEC // U1, l1, neg16)
        sm = _bmax_splat(m_c)[0]

        def rescan(st):
            def grp(g, st2):
                def gmax(i, m):
                    return jnp.maximum(m, buf[pl.ds(g * (GV * 16) + i * 16,
                                                    16)])
                mg = jax.lax.fori_loop(0, GV, gmax, neg16)
                sg = _bmax_splat(mg)[0]
                return jax.lax.cond(
                    sg > st2[2],
                    lambda s: extract_and_merge(buf, c, g, s),
                    lambda s: s, st2)
            return jax.lax.fori_loop(0, NG, grp, st)

        return jax.lax.cond(sm > st[2], rescan, lambda s: s, st)

    def row_body(k, carry):
        row = wid * rows_pw + k

        rbase = row * (NCH * CH)

        def start(c, slot):
            return pltpu.async_copy(
                x_hbm.at[pl.ds(rbase + c * CH, CH)], bufs[slot], sems[slot])

        st = (jnp.float32(NEG_INF), jnp.float32(NEG_INF),
              jnp.float32(NEG_INF), jnp.int32(0), jnp.int32(0), jnp.int32(0))
        hs = [start(0, 0), None]
        for c in range(NCH):
            if c + 1 < NCH:
                hs[(c + 1) % 2] = start(c + 1, (c + 1) % 2)
            hs[c % 2].wait()
            st = process_chunk(bufs[c % 2], c, st)

        t1, t2, t3, i1, i2, i3 = st
        resv[k] = jnp.where(iota == 0, t1,
                            jnp.where(iota == 1, t2,
                                      jnp.where(iota == 2, t3,
                                                jnp.float32(0.0))))
        resi[k] = jnp.where(iota == 0, i1,
                            jnp.where(iota == 1, i2,
                                      jnp.where(iota == 2, i3,
                                                jnp.int32(0))))
        return carry

    jax.lax.fori_loop(0, rows_pw, row_body, jnp.int32(0))
    pltpu.sync_copy(resv, topv_hbm.at[pl.ds(wid * rows_pw, rows_pw)])
    pltpu.sync_copy(resi, topi_hbm.at[pl.ds(wid * rows_pw, rows_pw)])


def _sc_topk(x2):
    n_rows, v = x2.shape
    assert v == NCH * CH
    xf = x2.reshape(n_rows * v)
    info = plsc.get_sparse_core_info()
    nw = info.num_cores * info.num_subcores
    rows_pw = n_rows // nw
    mesh = plsc.VectorSubcoreMesh(core_axis_name="c", subcore_axis_name="s")
    body = functools.partial(_sc_topk_body, rows_pw=rows_pw, nw=nw)
    fn = pl.kernel(
        body,
        mesh=mesh,
        out_type=[
            jax.ShapeDtypeStruct((n_rows, RES_W), jnp.float32),
            jax.ShapeDtypeStruct((n_rows, RES_W), jnp.int32),
        ],
        scratch_types=[
            pltpu.VMEM((CH,), jnp.float32),
            pltpu.VMEM((CH,), jnp.float32),
            pltpu.VMEM((rows_pw, RES_W), jnp.float32),
            pltpu.VMEM((rows_pw, RES_W), jnp.int32),
            pltpu.SemaphoreType.DMA,
            pltpu.SemaphoreType.DMA,
        ],
    )
    return fn(xf)


def _tc_entropy_body(x_ref, s_out, w_out, s_acc, w_acc, *, n_rows, cv, nv,
                     v_total):
    j = pl.program_id(0)

    @pl.when(j == 0)
    def _init():
        s_acc[...] = jnp.zeros_like(s_acc)
        w_acc[...] = jnp.zeros_like(w_acc)

    @pl.when(j < nv - 1)
    def _main():
        x = x_ref[...]
        e = jnp.exp(x)
        s_acc[...] += e
        w_acc[...] += x * e

    @pl.when(j == nv - 1)
    def _last():
        x = x_ref[...]
        col = j * cv + jax.lax.broadcasted_iota(jnp.int32, (n_rows, cv), 1)
        valid = col < v_total
        e = jnp.where(valid, jnp.exp(x), 0.0)
        s_acc[...] += e
        w_acc[...] += jnp.where(valid, x * e, 0.0)
        s_out[...] = jnp.sum(s_acc[...], axis=1, keepdims=True)
        w_out[...] = jnp.sum(w_acc[...], axis=1, keepdims=True)


def _tc_entropy(x2):
    n_rows, v = x2.shape
    cv = 2048
    nv = (v + cv - 1) // cv
    body = functools.partial(_tc_entropy_body, n_rows=n_rows, cv=cv, nv=nv,
                             v_total=v)
    return pl.pallas_call(
        body,
        grid=(nv,),
        in_specs=[pl.BlockSpec((n_rows, cv), lambda j: (0, j))],
        out_specs=[
            pl.BlockSpec((n_rows, 1), lambda j: (0, 0)),
            pl.BlockSpec((n_rows, 1), lambda j: (0, 0)),
        ],
        out_shape=[
            jax.ShapeDtypeStruct((n_rows, 1), jnp.float32),
            jax.ShapeDtypeStruct((n_rows, 1), jnp.float32),
        ],
        scratch_shapes=[
            pltpu.VMEM((n_rows, cv), jnp.float32),
            pltpu.VMEM((n_rows, cv), jnp.float32),
        ],
    )(x2)


def _assemble_body(ids_ref, params_ref, s_ref, w_ref, tv_ref, ti_ref,
                   out_idx_ref, out_prob_ref):
    S = s_ref[...]
    W = w_ref[...]
    ne = W / S - jnp.log(S)
    scale = params_ref[0, 0]
    centre = params_ref[0, 1]
    steep = params_ref[0, 2]
    ids = ids_ref[...]
    maskp = ids == MASK_ID
    lam = scale * jax.nn.sigmoid(steep * (ne - centre))
    lam = jnp.where(maskp, lam, 0.0)
    tv = tv_ref[:, 0:K]
    ti = jnp.where(maskp, ti_ref[:, 0:K], 0)
    et = jnp.exp(tv - jnp.max(tv, axis=1, keepdims=True))
    tp = et / jnp.sum(et, axis=1, keepdims=True)
    out_idx_ref[...] = jnp.concatenate([ids, ti], axis=1)
    out_prob_ref[...] = jnp.concatenate([1.0 - lam, lam * tp], axis=1)


def _assemble(ids2, params, S, W, topv, topi):
    n_rows = ids2.shape[0]
    return pl.pallas_call(
        _assemble_body,
        in_specs=[
            pl.BlockSpec((n_rows, 1), lambda: (0, 0)),
            pl.BlockSpec(memory_space=pltpu.SMEM),
            pl.BlockSpec((n_rows, 1), lambda: (0, 0)),
            pl.BlockSpec((n_rows, 1), lambda: (0, 0)),
            pl.BlockSpec((n_rows, RES_W), lambda: (0, 0)),
            pl.BlockSpec((n_rows, RES_W), lambda: (0, 0)),
        ],
        out_specs=[
            pl.BlockSpec((n_rows, 1 + K), lambda: (0, 0)),
            pl.BlockSpec((n_rows, 1 + K), lambda: (0, 0)),
        ],
        out_shape=[
            jax.ShapeDtypeStruct((n_rows, 1 + K), jnp.int32),
            jax.ShapeDtypeStruct((n_rows, 1 + K), jnp.float32),
        ],
    )(ids2, params, S, W, topv, topi)


def kernel(input_ids, logits_prelim, raw_scale, raw_centre_neg, raw_steep,
           raw_temperature):
    B, T, V = logits_prelim.shape
    n_rows = B * T

    x2 = logits_prelim.reshape(n_rows, V)
    ids2 = input_ids.reshape(n_rows, 1).astype(jnp.int32)
    scale = jax.nn.sigmoid(raw_scale)
    centre = -jax.nn.softplus(raw_centre_neg) - EPS
    steep = jax.nn.softplus(raw_steep) + EPS
    params = jnp.stack([scale, centre, steep]).reshape(1, 3)

    topv, topi = _sc_topk(x2)
    S, W = _tc_entropy(x2)
    out_idx, out_prob = _assemble(ids2, params, S, W, topv, topi)

    final_indices = out_idx.reshape(B, T, 1 + K)
    final_probs = out_prob.reshape(B, T, 1 + K)
    return final_indices, final_probs


# speedup vs baseline: 1.0037x; 1.0037x over previous
import functools

import jax
import jax.numpy as jnp
from jax.experimental import pallas as pl
from jax.experimental.pallas import tpu as pltpu
from jax.experimental.pallas import tpu_sc as plsc

MASK_ID = 5
K = 3
EPS = 1e-06
NEG_INF = float("-inf")
I32_BIG = jnp.iinfo(jnp.int32).max

CH = 10000
NCH = 10
NVEC = CH // 16
U1 = 25
GV = 25
NG = NVEC // GV
RES_W = 16


def _bmax_splat(v):
    i = jax.lax.broadcasted_iota(jnp.int32, (16,), 0)
    for d in (1, 2, 4, 8):
        v = jnp.maximum(v, v[i ^ d])
    return v


def _bmin_splat_i32(v):
    i = jax.lax.broadcasted_iota(jnp.int32, (16,), 0)
    for d in (1, 2, 4, 8):
        v = jnp.minimum(v, v[i ^ d])
    return v


def _sc_topk_body(x_hbm, topv_hbm, topi_hbm, buf0, buf1, resv, resi,
                  sem0, sem1, *, rows_pw, nw):
    nc = 2
    wid = jax.lax.axis_index("s") * nc + jax.lax.axis_index("c")
    iota = jax.lax.broadcasted_iota(jnp.int32, (16,), 0)
    neg16 = jnp.full((16,), NEG_INF, jnp.float32)
    big16 = jnp.full((16,), I32_BIG, jnp.int32)
    bufs = (buf0, buf1)
    sems = (sem0, sem1)

    def extract_and_merge(buf, c, g, st):
        b0 = c * CH + g * (GV * 16)
        lb = g * (GV * 16)
        rem1 = jnp.int32(-1)
        rem2 = jnp.int32(-1)
        cand = []
        for _ in range(K):
            def maxpass(i, m, rem1=rem1, rem2=rem2):
                v = buf[pl.ds(lb + i * 16, 16)]
                gi = (b0 + i * 16) + iota
                keep = (gi != rem1) & (gi != rem2)
                return jnp.maximum(m, jnp.where(keep, v, NEG_INF))

            mvec = jax.lax.fori_loop(0, GV, maxpass, neg16)
            mv = _bmax_splat(mvec)[0]

            def ipass(i, iv, mv=mv, rem1=rem1, rem2=rem2):
                v = buf[pl.ds(lb + i * 16, 16)]
                gi = (b0 + i * 16) + iota
                ok = (v == mv) & (gi != rem1) & (gi != rem2)
                return jnp.minimum(iv, jnp.where(ok, gi, I32_BIG))

            ivec = jax.lax.fori_loop(0, GV, ipass, big16)
            mi = _bmin_splat_i32(ivec)[0]
            cand.append((mv, mi))
            rem2 = rem1
            rem1 = mi

        vs = [st[0], st[1], st[2], cand[0][0], cand[1][0], cand[2][0]]
        ix = [st[3], st[4], st[5], cand[0][1], cand[1][1], cand[2][1]]
        outv, outi = [], []
        for _ in range(K):
            bv, bi = vs[0], ix[0]
            for t in range(1, 6):
                better = (vs[t] > bv) | ((vs[t] == bv) & (ix[t] < bi))
                bv = jnp.where(better, vs[t], bv)
                bi = jnp.where(better, ix[t], bi)
            outv.append(bv)
            outi.append(bi)
            vs = [jnp.where((vs[t] == bv) & (ix[t] == bi),
                            jnp.float32(NEG_INF), vs[t]) for t in range(6)]
        return (outv[0], outv[1], outv[2], outi[0], outi[1], outi[2])

    NCHAIN = 5

    def process_chunk(buf, c, st):
        def l1(i, ms):
            ms = list(ms)
            for u in range(U1):
                ms[u % NCHAIN] = jnp.maximum(
                    ms[u % NCHAIN], buf[pl.ds((i * U1 + u) * 16, 16)])
            return tuple(ms)

        m5 = jax.lax.fori_loop(0, NVEC // U1, l1, (neg16,) * NCHAIN)
        m_c = m5[0]
        for t in range(1, NCHAIN):
            m_c = jnp.maximum(m_c, m5[t])
        sm = _bmax_splat(m_c)[0]

        def rescan(st):
            def grp(g, st2):
                gb = g * (GV * 16)
                mgs = [neg16] * NCHAIN
                for i in range(GV):
                    mgs[i % NCHAIN] = jnp.maximum(
                        mgs[i % NCHAIN], buf[pl.ds(gb + i * 16, 16)])
                mg = mgs[0]
                for t in range(1, NCHAIN):
                    mg = jnp.maximum(mg, mgs[t])
                sg = _bmax_splat(mg)[0]
                return jax.lax.cond(
                    sg > st2[2],
                    lambda s: extract_and_merge(buf, c, g, s),
                    lambda s: s, st2)
            return jax.lax.fori_loop(0, NG, grp, st)

        return jax.lax.cond(sm > st[2], rescan, lambda s: s, st)

    def row_body(k, carry):
        row = wid * rows_pw + k

        rbase = row * (NCH * CH)

        def start(c, slot):
            return pltpu.async_copy(
                x_hbm.at[pl.ds(rbase + c * CH, CH)], bufs[slot], sems[slot])

        st = (jnp.float32(NEG_INF), jnp.float32(NEG_INF),
              jnp.float32(NEG_INF), jnp.int32(0), jnp.int32(0), jnp.int32(0))
        hs = [start(0, 0), None]
        for c in range(NCH):
            if c + 1 < NCH:
                hs[(c + 1) % 2] = start(c + 1, (c + 1) % 2)
            hs[c % 2].wait()
            st = process_chunk(bufs[c % 2], c, st)

        t1, t2, t3, i1, i2, i3 = st
        resv[k] = jnp.where(iota == 0, t1,
                            jnp.where(iota == 1, t2,
                                      jnp.where(iota == 2, t3,
                                                jnp.float32(0.0))))
        resi[k] = jnp.where(iota == 0, i1,
                            jnp.where(iota == 1, i2,
                                      jnp.where(iota == 2, i3,
                                                jnp.int32(0))))
        return carry

    jax.lax.fori_loop(0, rows_pw, row_body, jnp.int32(0))
    pltpu.sync_copy(resv, topv_hbm.at[pl.ds(wid * rows_pw, rows_pw)])
    pltpu.sync_copy(resi, topi_hbm.at[pl.ds(wid * rows_pw, rows_pw)])


def _sc_topk(x2):
    n_rows, v = x2.shape
    assert v == NCH * CH
    xf = x2.reshape(n_rows * v)
    info = plsc.get_sparse_core_info()
    nw = info.num_cores * info.num_subcores
    rows_pw = n_rows // nw
    mesh = plsc.VectorSubcoreMesh(core_axis_name="c", subcore_axis_name="s")
    body = functools.partial(_sc_topk_body, rows_pw=rows_pw, nw=nw)
    fn = pl.kernel(
        body,
        mesh=mesh,
        out_type=[
            jax.ShapeDtypeStruct((n_rows, RES_W), jnp.float32),
            jax.ShapeDtypeStruct((n_rows, RES_W), jnp.int32),
        ],
        scratch_types=[
            pltpu.VMEM((CH,), jnp.float32),
            pltpu.VMEM((CH,), jnp.float32),
            pltpu.VMEM((rows_pw, RES_W), jnp.float32),
            pltpu.VMEM((rows_pw, RES_W), jnp.int32),
            pltpu.SemaphoreType.DMA,
            pltpu.SemaphoreType.DMA,
        ],
    )
    return fn(xf)


def _tc_entropy_body(x_ref, s_out, w_out, s_acc, w_acc, *, n_rows, cv, nv,
                     v_total):
    j = pl.program_id(0)

    @pl.when(j == 0)
    def _init():
        s_acc[...] = jnp.zeros_like(s_acc)
        w_acc[...] = jnp.zeros_like(w_acc)

    @pl.when(j < nv - 1)
    def _main():
        x = x_ref[...]
        e = jnp.exp(x)
        s_acc[...] += e
        w_acc[...] += x * e

    @pl.when(j == nv - 1)
    def _last():
        x = x_ref[...]
        col = j * cv + jax.lax.broadcasted_iota(jnp.int32, (n_rows, cv), 1)
        valid = col < v_total
        e = jnp.where(valid, jnp.exp(x), 0.0)
        s_acc[...] += e
        w_acc[...] += jnp.where(valid, x * e, 0.0)
        s_out[...] = jnp.sum(s_acc[...], axis=1, keepdims=True)
        w_out[...] = jnp.sum(w_acc[...], axis=1, keepdims=True)


def _tc_entropy(x2):
    n_rows, v = x2.shape
    cv = 2048
    nv = (v + cv - 1) // cv
    body = functools.partial(_tc_entropy_body, n_rows=n_rows, cv=cv, nv=nv,
                             v_total=v)
    return pl.pallas_call(
        body,
        grid=(nv,),
        in_specs=[pl.BlockSpec((n_rows, cv), lambda j: (0, j))],
        out_specs=[
            pl.BlockSpec((n_rows, 1), lambda j: (0, 0)),
            pl.BlockSpec((n_rows, 1), lambda j: (0, 0)),
        ],
        out_shape=[
            jax.ShapeDtypeStruct((n_rows, 1), jnp.float32),
            jax.ShapeDtypeStruct((n_rows, 1), jnp.float32),
        ],
        scratch_shapes=[
            pltpu.VMEM((n_rows, cv), jnp.float32),
            pltpu.VMEM((n_rows, cv), jnp.float32),
        ],
    )(x2)


def _assemble_body(ids_ref, params_ref, s_ref, w_ref, tv_ref, ti_ref,
                   out_idx_ref, out_prob_ref):
    S = s_ref[...]
    W = w_ref[...]
    ne = W / S - jnp.log(S)
    scale = params_ref[0, 0]
    centre = params_ref[0, 1]
    steep = params_ref[0, 2]
    ids = ids_ref[...]
    maskp = ids == MASK_ID
    lam = scale * jax.nn.sigmoid(steep * (ne - centre))
    lam = jnp.where(maskp, lam, 0.0)
    tv = tv_ref[:, 0:K]
    ti = jnp.where(maskp, ti_ref[:, 0:K], 0)
    et = jnp.exp(tv - jnp.max(tv, axis=1, keepdims=True))
    tp = et / jnp.sum(et, axis=1, keepdims=True)
    out_idx_ref[...] = jnp.concatenate([ids, ti], axis=1)
    out_prob_ref[...] = jnp.concatenate([1.0 - lam, lam * tp], axis=1)


def _assemble(ids2, params, S, W, topv, topi):
    n_rows = ids2.shape[0]
    return pl.pallas_call(
        _assemble_body,
        in_specs=[
            pl.BlockSpec((n_rows, 1), lambda: (0, 0)),
            pl.BlockSpec(memory_space=pltpu.SMEM),
            pl.BlockSpec((n_rows, 1), lambda: (0, 0)),
            pl.BlockSpec((n_rows, 1), lambda: (0, 0)),
            pl.BlockSpec((n_rows, RES_W), lambda: (0, 0)),
            pl.BlockSpec((n_rows, RES_W), lambda: (0, 0)),
        ],
        out_specs=[
            pl.BlockSpec((n_rows, 1 + K), lambda: (0, 0)),
            pl.BlockSpec((n_rows, 1 + K), lambda: (0, 0)),
        ],
        out_shape=[
            jax.ShapeDtypeStruct((n_rows, 1 + K), jnp.int32),
            jax.ShapeDtypeStruct((n_rows, 1 + K), jnp.float32),
        ],
    )(ids2, params, S, W, topv, topi)


def kernel(input_ids, logits_prelim, raw_scale, raw_centre_neg, raw_steep,
           raw_temperature):
    B, T, V = logits_prelim.shape
    n_rows = B * T

    x2 = logits_prelim.reshape(n_rows, V)
    ids2 = input_ids.reshape(n_rows, 1).astype(jnp.int32)
    scale = jax.nn.sigmoid(raw_scale)
    centre = -jax.nn.softplus(raw_centre_neg) - EPS
    steep = jax.nn.softplus(raw_steep) + EPS
    params = jnp.stack([scale, centre, steep]).reshape(1, 3)

    topv, topi = _sc_topk(x2)
    S, W = _tc_entropy(x2)
    out_idx, out_prob = _assemble(ids2, params, S, W, topv, topi)

    final_indices = out_idx.reshape(B, T, 1 + K)
    final_probs = out_prob.reshape(B, T, 1 + K)
    return final_indices, final_probs
